# Initial kernel scaffold; baseline (speedup 1.0000x reference)
import jax
import jax.numpy as jnp
from jax.experimental import pallas as pl


def kernel(x, edge_index, batch, params):
    raise NotImplementedError("write your pallas kernel here")


# speedup vs baseline: 3.9373x; 3.9373x over previous
import functools
import math

import jax
import jax.numpy as jnp
from jax import lax
from jax.experimental import pallas as pl
from jax.experimental.pallas import tpu as pltpu
from jax.experimental.pallas import tpu_sc as plsc

N = 10000
D = 128
G = 64
L_OUT = 64

NC = 2
NS = 16
NW = NC * NS
CHUNK = 128
N_PAD = 10112
ROWS_PER_TILE = N_PAD // NS


def _make_sc_agg(cpw):
    mesh = plsc.VectorSubcoreMesh(core_axis_name="c", subcore_axis_name="s")

    @functools.partial(
        pl.kernel,
        mesh=mesh,
        out_type=jax.ShapeDtypeStruct((NC * N_PAD, D), jnp.float32),
        scratch_types=[
            pltpu.VMEM((CHUNK,), jnp.int32),
            pltpu.VMEM((CHUNK,), jnp.int32),
            pltpu.VMEM((CHUNK, D), jnp.float32),
            pltpu.VMEM_SHARED((N_PAD, D), jnp.float32),
            pltpu.SemaphoreType.DMA,
        ],
    )
    def sc_agg(h_hbm, src_hbm, dst_hbm, zeros_hbm, out_hbm,
               src_v, dst_v, rows_v, acc_sh, sem):
        c = lax.axis_index("c")
        s = lax.axis_index("s")
        wid = s * NC + c
        r0 = s * ROWS_PER_TILE
        pltpu.sync_copy(zeros_hbm.at[pl.ds(r0, ROWS_PER_TILE)],
                        acc_sh.at[pl.ds(r0, ROWS_PER_TILE)])
        plsc.subcore_barrier()

        def body(j, carry):
            off = (wid * cpw + j) * CHUNK
            pltpu.sync_copy(src_hbm.at[pl.ds(off, CHUNK)], src_v)
            pltpu.sync_copy(dst_hbm.at[pl.ds(off, CHUNK)], dst_v)
            pltpu.async_copy(h_hbm.at[src_v], rows_v, sem).wait()
            pltpu.sync_copy(rows_v, acc_sh.at[dst_v], add=True)
            return carry

        lax.fori_loop(0, cpw, body, 0)
        plsc.subcore_barrier()
        pltpu.sync_copy(acc_sh.at[pl.ds(r0, ROWS_PER_TILE)],
                        out_hbm.at[pl.ds(c * N_PAD + r0, ROWS_PER_TILE)])

    return sc_agg


BLK = 1000


def _gelu(x):
    return 0.5 * x * (1.0 + lax.erf(x * (1.0 / math.sqrt(2.0))))


def _affine_body(x_ref, s_ref, t_ref, o_ref):
    o_ref[...] = x_ref[...] * s_ref[...] + t_ref[...]


_affine_call = pl.pallas_call(
    _affine_body,
    grid=(N // BLK,),
    in_specs=[
        pl.BlockSpec((BLK, D), lambda i: (i, 0)),
        pl.BlockSpec((1, D), lambda i: (0, 0)),
        pl.BlockSpec((1, D), lambda i: (0, 0)),
    ],
    out_specs=pl.BlockSpec((BLK, D), lambda i: (i, 0)),
    out_shape=jax.ShapeDtypeStruct((N, D), jnp.float32),
)


def _layer_body(h_ref, a0_ref, a1_ref, epsr_ref, w1_ref, b1_ref,
                w2_ref, b2_ref, s2_ref, t2_ref, o_ref):
    m = h_ref[...] * epsr_ref[...] + a0_ref[0] + a1_ref[0]
    y = _gelu(jnp.dot(m, w1_ref[...], preferred_element_type=jnp.float32)
              + b1_ref[...])
    z = jnp.dot(y, w2_ref[...], preferred_element_type=jnp.float32) + b2_ref[...]
    o_ref[...] = _gelu(z * s2_ref[...] + t2_ref[...])


_layer_call = pl.pallas_call(
    _layer_body,
    grid=(N // BLK,),
    in_specs=[
        pl.BlockSpec((BLK, D), lambda i: (i, 0)),
        pl.BlockSpec((1, BLK, D), lambda i: (0, i, 0)),
        pl.BlockSpec((1, BLK, D), lambda i: (1, i, 0)),
        pl.BlockSpec((1, D), lambda i: (0, 0)),
        pl.BlockSpec((D, D), lambda i: (0, 0)),
        pl.BlockSpec((1, D), lambda i: (0, 0)),
        pl.BlockSpec((D, D), lambda i: (0, 0)),
        pl.BlockSpec((1, D), lambda i: (0, 0)),
        pl.BlockSpec((1, D), lambda i: (0, 0)),
        pl.BlockSpec((1, D), lambda i: (0, 0)),
    ],
    out_specs=pl.BlockSpec((BLK, D), lambda i: (i, 0)),
    out_shape=jax.ShapeDtypeStruct((N, D), jnp.float32),
)


def _pool_head_body(h_ref, b_ref, w1_ref, b1_ref, lg_ref, lb_ref,
                    w2_ref, b2_ref, o_ref, pool_acc, cnt_acc):
    i = pl.program_id(0)

    @pl.when(i == 0)
    def _():
        pool_acc[...] = jnp.zeros_like(pool_acc)
        cnt_acc[...] = jnp.zeros_like(cnt_acc)

    mask = (b_ref[...] == lax.broadcasted_iota(jnp.int32, (BLK, G), 1)
            ).astype(jnp.float32)
    pool_acc[...] += lax.dot_general(mask, h_ref[...],
                                     (((0,), (0,)), ((), ())),
                                     preferred_element_type=jnp.float32)
    cnt_acc[...] += lax.dot_general(mask, jnp.ones((BLK, 1), jnp.float32),
                                    (((0,), (0,)), ((), ())),
                                    preferred_element_type=jnp.float32)

    @pl.when(i == pl.num_programs(0) - 1)
    def _():
        cnt = jnp.maximum(cnt_acc[...], 1.0)
        pooled = pool_acc[...] / cnt
        o1 = jnp.dot(pooled, w1_ref[...],
                     preferred_element_type=jnp.float32) + b1_ref[...]
        mu = jnp.mean(o1, axis=-1, keepdims=True)
        var = jnp.mean((o1 - mu) ** 2, axis=-1, keepdims=True)
        o1 = (o1 - mu) / jnp.sqrt(var + 1e-5) * lg_ref[...] + lb_ref[...]
        o1 = _gelu(o1) + pooled
        o_ref[...] = jnp.dot(o1, w2_ref[...],
                             preferred_element_type=jnp.float32) + b2_ref[...]


_pool_head_call = pl.pallas_call(
    _pool_head_body,
    grid=(N // BLK,),
    in_specs=[
        pl.BlockSpec((BLK, D), lambda i: (i, 0)),
        pl.BlockSpec((BLK, 1), lambda i: (i, 0)),
        pl.BlockSpec((D, D), lambda i: (0, 0)),
        pl.BlockSpec((1, D), lambda i: (0, 0)),
        pl.BlockSpec((1, D), lambda i: (0, 0)),
        pl.BlockSpec((1, D), lambda i: (0, 0)),
        pl.BlockSpec((D, L_OUT), lambda i: (0, 0)),
        pl.BlockSpec((1, L_OUT), lambda i: (0, 0)),
    ],
    out_specs=pl.BlockSpec((G, L_OUT), lambda i: (0, 0)),
    out_shape=jax.ShapeDtypeStruct((G, L_OUT), jnp.float32),
    scratch_shapes=[
        pltpu.VMEM((G, D), jnp.float32),
        pltpu.VMEM((G, 1), jnp.float32),
    ],
)


def kernel(x, edge_index, batch, params):
    src = edge_index[0].astype(jnp.int32)
    dst = edge_index[1].astype(jnp.int32)
    e = src.shape[0]
    cpw = -(-e // (NW * CHUNK))
    e_pad = NW * cpw * CHUNK
    pad = e_pad - e
    src = jnp.concatenate([src, jnp.zeros((pad,), jnp.int32)])
    dst = jnp.concatenate([dst, jnp.full((pad,), N, jnp.int32)])
    zeros = jnp.zeros((N_PAD, D), jnp.float32)
    bi = batch.astype(jnp.int32).reshape(N, 1)

    sc_agg = _make_sc_agg(cpw)

    p_in = params["input_bn"]
    s_in = (p_in["gamma"] / jnp.sqrt(p_in["var"] + 1e-5)).reshape(1, D)
    t_in = (p_in["beta"] - p_in["mean"] * s_in[0]).reshape(1, D)
    h = _affine_call(x, s_in, t_in)

    for cp in params["convs"]:
        mlp = cp["mlp"]
        bn1 = mlp["bn1"]
        s1 = bn1["gamma"] / jnp.sqrt(bn1["var"] + 1e-5)
        t1 = bn1["beta"] - bn1["mean"] * s1
        w1f = mlp["fc1_w"] * s1[None, :]
        b1f = (mlp["fc1_b"] * s1 + t1).reshape(1, D)
        w2 = mlp["fc2_w"]
        b2 = mlp["fc2_b"].reshape(1, D)
        bn = cp["bn"]
        s2 = (bn["gamma"] / jnp.sqrt(bn["var"] + 1e-5)).reshape(1, D)
        t2 = (bn["beta"] - bn["mean"] * s2[0]).reshape(1, D)
        epsr = jnp.full((1, D), 1.0, jnp.float32) + cp["eps"]

        parts = sc_agg(h, src, dst, zeros).reshape(NC, N_PAD, D)
        h = _layer_call(h, parts, parts, epsr, w1f, b1f, w2, b2, s2, t2)

    return _pool_head_call(
        h, bi,
        params["fc1_w"], params["fc1_b"].reshape(1, D),
        params["ln_gamma"].reshape(1, D), params["ln_beta"].reshape(1, D),
        params["fc2_w"], params["fc2_b"].reshape(1, L_OUT),
    )
